# baseline (speedup 1.0000x reference)
import functools

import jax
import jax.numpy as jnp
from jax.experimental import pallas as pl
from jax.experimental.pallas import tpu as pltpu

Q = 1024
D = 128
K = 100000
KBLK = 1024
QBLK = 256
NKB = (K + KBLK - 1) // KBLK
NQB = Q // QBLK


def _nn_kernel(qt_ref, v_ref, out_ref, minval, minidx):
    ki = pl.program_id(0)
    qi = pl.program_id(1)
    qt = qt_ref[...]
    v = v_ref[...]
    dot = jax.lax.dot_general(
        v, qt, (((1,), (0,)), ((), ())),
        preferred_element_type=jnp.float32,
    )
    v2 = jnp.sum(v * v, axis=1, keepdims=True)
    score = v2 - 2.0 * dot
    sub = jax.lax.broadcasted_iota(jnp.int32, (KBLK, 1), 0)
    gidx = ki * KBLK + sub
    score = jnp.where(gidx < K, score, jnp.inf)

    local_min = jnp.min(score, axis=0, keepdims=True)
    big = jnp.int32(K + KBLK)
    cand = jnp.where(score <= local_min, gidx, big)
    local_arg = jnp.min(cand, axis=0, keepdims=True)

    col = qi * QBLK

    @pl.when(ki == 0)
    def _init():
        minval[:, pl.ds(col, QBLK)] = local_min
        minidx[:, pl.ds(col, QBLK)] = local_arg

    @pl.when(ki != 0)
    def _update():
        prev_val = minval[:, pl.ds(col, QBLK)]
        prev_idx = minidx[:, pl.ds(col, QBLK)]
        better = local_min < prev_val
        minval[:, pl.ds(col, QBLK)] = jnp.where(better, local_min, prev_val)
        minidx[:, pl.ds(col, QBLK)] = jnp.where(better, local_arg, prev_idx)

    @pl.when(ki == NKB - 1)
    def _emit():
        out_ref[...] = minidx[:, pl.ds(col, QBLK)]


@functools.partial(jax.jit, static_argnames=())
def kernel(query, vectors):
    if query.ndim == 1:
        query = query[None, :]
    qt = query.T
    out = pl.pallas_call(
        _nn_kernel,
        grid=(NKB, NQB),
        in_specs=[
            pl.BlockSpec((D, QBLK), lambda ki, qi: (0, qi)),
            pl.BlockSpec((KBLK, D), lambda ki, qi: (ki, 0)),
        ],
        out_specs=pl.BlockSpec((1, QBLK), lambda ki, qi: (0, qi)),
        out_shape=jax.ShapeDtypeStruct((1, Q), jnp.int32),
        scratch_shapes=[
            pltpu.VMEM((1, Q), jnp.float32),
            pltpu.VMEM((1, Q), jnp.int32),
        ],
        compiler_params=pltpu.CompilerParams(
            dimension_semantics=("arbitrary", "arbitrary"),
        ),
    )(qt, vectors)
    return out[0]


# speedup vs baseline: 1.9844x; 1.9844x over previous
import functools

import jax
import jax.numpy as jnp
from jax.experimental import pallas as pl
from jax.experimental.pallas import tpu as pltpu

Q = 1024
D = 128
K = 100000
KBLK = 1024
NKB = K // KBLK
TAIL = K - NKB * KBLK


def _nn_main(qt2_ref, v_ref, val_out, idx_out, minval, minidx):
    ki = pl.program_id(0)
    qt2 = qt2_ref[...]
    v = v_ref[...]
    dot = jax.lax.dot_general(
        v, qt2, (((1,), (0,)), ((), ())),
        preferred_element_type=jnp.float32,
    )
    v2 = jnp.sum(v * v, axis=1, keepdims=True)
    score = dot + v2

    sub = jax.lax.broadcasted_iota(jnp.int32, (KBLK, 1), 0)
    gidx = ki * KBLK + sub

    local_min = jnp.min(score, axis=0, keepdims=True)
    big = jnp.int32(K)
    cand = jnp.where(score <= local_min, gidx, big)
    local_arg = jnp.min(cand, axis=0, keepdims=True)

    @pl.when(ki == 0)
    def _init():
        minval[...] = local_min
        minidx[...] = local_arg

    @pl.when(ki != 0)
    def _update():
        prev_val = minval[...]
        prev_idx = minidx[...]
        better = local_min < prev_val
        minval[...] = jnp.where(better, local_min, prev_val)
        minidx[...] = jnp.where(better, local_arg, prev_idx)

    @pl.when(ki == NKB - 1)
    def _emit():
        val_out[...] = minval[...]
        idx_out[...] = minidx[...]


def _nn_tail(vt_ref, qt2_ref, pval_ref, pidx_ref, out_ref):
    vt = vt_ref[...]
    qt2 = qt2_ref[...]
    dot = jax.lax.dot_general(
        vt, qt2, (((1,), (0,)), ((), ())),
        preferred_element_type=jnp.float32,
    )
    v2 = jnp.sum(vt * vt, axis=1, keepdims=True)
    score = dot + v2
    sub = jax.lax.broadcasted_iota(jnp.int32, (TAIL, 1), 0)
    gidx = NKB * KBLK + sub

    local_min = jnp.min(score, axis=0, keepdims=True)
    big = jnp.int32(K)
    cand = jnp.where(score <= local_min, gidx, big)
    local_arg = jnp.min(cand, axis=0, keepdims=True)

    better = local_min < pval_ref[...]
    out_ref[...] = jnp.where(better, local_arg, pidx_ref[...])


@functools.partial(jax.jit, static_argnames=())
def kernel(query, vectors):
    if query.ndim == 1:
        query = query[None, :]
    qt2 = (-2.0 * query).T
    vmain = vectors
    vtail = vectors[NKB * KBLK:]

    pval, pidx = pl.pallas_call(
        _nn_main,
        grid=(NKB,),
        in_specs=[
            pl.BlockSpec((D, Q), lambda ki: (0, 0)),
            pl.BlockSpec((KBLK, D), lambda ki: (ki, 0)),
        ],
        out_specs=[
            pl.BlockSpec((1, Q), lambda ki: (0, 0)),
            pl.BlockSpec((1, Q), lambda ki: (0, 0)),
        ],
        out_shape=[
            jax.ShapeDtypeStruct((1, Q), jnp.float32),
            jax.ShapeDtypeStruct((1, Q), jnp.int32),
        ],
        scratch_shapes=[
            pltpu.VMEM((1, Q), jnp.float32),
            pltpu.VMEM((1, Q), jnp.int32),
        ],
        compiler_params=pltpu.CompilerParams(
            dimension_semantics=("arbitrary",),
        ),
    )(qt2, vmain)

    out = pl.pallas_call(
        _nn_tail,
        out_shape=jax.ShapeDtypeStruct((1, Q), jnp.int32),
    )(vtail, qt2, pval, pidx)
    return out[0]
